# Initial kernel scaffold; baseline (speedup 1.0000x reference)
import jax
import jax.numpy as jnp
from jax.experimental import pallas as pl


def kernel(x, edge_index):
    raise NotImplementedError("write your pallas kernel here")


# speedup vs baseline: 3.8439x; 3.8439x over previous
import functools

import numpy as np
import jax
import jax.numpy as jnp
from jax import lax
from jax.experimental import pallas as pl
from jax.experimental.pallas import tpu as pltpu
from jax.experimental.pallas import tpu_sc as plsc

N_NODES = 10000
N_TILES = 32
RPT = 320
NPAD = N_TILES * RPT
E = 320000
C = 2000
NCHUNK = E // C
NG16 = C // 16
G = 128
FBUF = 2048

NUM_PERM = 128
HLL_M = 256
MAX_RANK = 24
ALPHA = 0.7213 / (1.0 + 1.079 / HLL_M)


def _init_tables():
    rng = np.random.RandomState(1)
    perm_a = rng.randint(1, 2 ** 30, size=(1, NUM_PERM)).astype(np.int64)
    perm_b = rng.randint(0, 2 ** 30, size=(1, NUM_PERM)).astype(np.int64)
    ids = np.arange(N_NODES, dtype=np.int64)
    h = (ids * np.int64(2654435761)) & np.int64(0xFFFFFFFF)
    mh = ((perm_a * h[:, None] + perm_b) % np.int64((1 << 61) - 1)) \
        & np.int64((1 << 32) - 1)
    mh32 = (mh - np.int64(1 << 31)).astype(np.int32)
    mh32 = np.concatenate(
        [mh32, np.zeros((NPAD - N_NODES, NUM_PERM), np.int32)], axis=0)

    reg_index = (h & np.int64(HLL_M - 1)).astype(np.int64)
    bits = h >> 8
    bit_len = np.where(bits > 0,
                       np.floor(np.log2(np.maximum(bits, 1).astype(np.float64)))
                       .astype(np.int64) + 1, 0)
    ranks = (MAX_RANK - bit_len + 1).astype(np.int8)
    regs = np.zeros((NPAD, HLL_M), dtype=np.int8)
    regs[np.arange(N_NODES), reg_index] = ranks
    return mh32, regs


_MH0_NP, _HLL0_NP = _init_tables()


def _hop_body(mh_tab, hll_tab, src_hbm, dst_hbm, mh_out, hll_out,
              acc_mh, acc_hll, srcb, dstb, fsrc, fdst, rows_mh, rows_hll,
              sem):
    i32 = jnp.int32
    cid = lax.axis_index("c").astype(i32)
    sid = lax.axis_index("s").astype(i32)
    wid = sid * i32(2) + cid
    lo = wid * i32(RPT)

    pltpu.sync_copy(mh_tab.at[pl.ds(lo, RPT)], acc_mh)
    pltpu.sync_copy(hll_tab.at[pl.ds(lo, RPT)], acc_hll)

    def zfill(k, carry):
        fsrc[pl.ds(k * i32(16), 16)] = jnp.zeros((16,), jnp.int32)
        return carry
    lax.fori_loop(i32(0), i32(FBUF // 16), zfill, i32(0))

    def chunk_body(c, carry):
        pltpu.sync_copy(src_hbm.at[pl.ds(c * i32(C), C)], srcb)
        pltpu.sync_copy(dst_hbm.at[pl.ds(c * i32(C), C)], dstb)

        def filt(g, cnt):
            s16 = srcb[pl.ds(g * i32(16), 16)]
            d16 = dstb[pl.ds(g * i32(16), 16)]
            dl = d16 - lo
            m = (dl >= i32(0)) & (dl < i32(RPT))
            mi = m.astype(jnp.int32)
            pos = jnp.cumsum(mi) - mi + cnt
            plsc.store_scatter(fsrc, [pos], s16, mask=m)
            plsc.store_scatter(fdst, [pos], dl, mask=m)
            return cnt + jnp.sum(mi, dtype=jnp.int32)

        k_edges = lax.fori_loop(i32(0), i32(NG16), filt, i32(0))
        ngroups = (k_edges + i32(G - 1)) // i32(G)

        def grp(g, carry2):
            idx = fsrc.at[pl.ds(g * i32(G), G)]
            pltpu.async_copy(mh_tab.at[idx], rows_mh, sem).wait()
            pltpu.async_copy(hll_tab.at[idx], rows_hll, sem).wait()

            def apply(i, carry3):
                d = fdst[pl.ds(i, 16)][0]
                r = i - g * i32(G)
                for sub in range(NUM_PERM // 16):
                    sl = pl.ds(sub * 16, 16)
                    acc_mh[d, sl] = jnp.minimum(acc_mh[d, sl], rows_mh[r, sl])
                for sub in range(HLL_M // 64):
                    sl = pl.ds(sub * 16, 16)
                    a8 = plsc.bitcast(acc_hll[d, sl], jnp.int8)
                    r8 = plsc.bitcast(rows_hll[r, sl], jnp.int8)
                    acc_hll[d, sl] = plsc.bitcast(jnp.maximum(a8, r8),
                                                  jnp.int32)
                return carry3

            lax.fori_loop(g * i32(G),
                          jnp.minimum(k_edges, (g + i32(1)) * i32(G)),
                          apply, i32(0))
            return carry2

        lax.fori_loop(i32(0), ngroups, grp, i32(0))
        return carry

    lax.fori_loop(i32(0), i32(NCHUNK), chunk_body, i32(0))

    pltpu.sync_copy(acc_mh, mh_out.at[pl.ds(lo, RPT)])
    pltpu.sync_copy(acc_hll, hll_out.at[pl.ds(lo, RPT)])


_hop = functools.partial(
    pl.kernel,
    mesh=plsc.VectorSubcoreMesh(core_axis_name="c", subcore_axis_name="s"),
    compiler_params=pltpu.CompilerParams(needs_layout_passes=False,
                                         use_tc_tiling_on_sc=False),
    out_type=[
        jax.ShapeDtypeStruct((NPAD, NUM_PERM), jnp.int32),
        jax.ShapeDtypeStruct((NPAD, HLL_M // 4), jnp.int32),
    ],
    scratch_types=[
        pltpu.VMEM((RPT, NUM_PERM), jnp.int32),
        pltpu.VMEM((RPT, HLL_M // 4), jnp.int32),
        pltpu.VMEM((C,), jnp.int32),
        pltpu.VMEM((C,), jnp.int32),
        pltpu.VMEM((FBUF,), jnp.int32),
        pltpu.VMEM((FBUF,), jnp.int32),
        pltpu.VMEM((G, NUM_PERM), jnp.int32),
        pltpu.VMEM((G, HLL_M // 4), jnp.int32),
        pltpu.SemaphoreType.DMA,
    ],
)(_hop_body)


def _cards_body(h1_ref, h2_ref, o1_ref, o2_ref):
    for href, oref in ((h1_ref, o1_ref), (h2_ref, o2_ref)):
        r = href[...].astype(jnp.float32)
        e = ALPHA * (HLL_M * HLL_M) / jnp.sum(jnp.exp2(-r), axis=1)
        v = jnp.sum((href[...] == 0).astype(jnp.float32), axis=1)
        lc = HLL_M * jnp.log(HLL_M / jnp.maximum(v, 1.0))
        oref[...] = jnp.where((e <= 2.5 * HLL_M) & (v > 0), lc, e)


def _cards(h1, h2):
    return pl.pallas_call(
        _cards_body,
        out_shape=[
            jax.ShapeDtypeStruct((NPAD,), jnp.float32),
            jax.ShapeDtypeStruct((NPAD,), jnp.float32),
        ],
    )(h1, h2)


def _pack8(a):
    return lax.bitcast_convert_type(
        a.reshape(NPAD, HLL_M // 4, 4), jnp.int32)


def _unpack8(a):
    return lax.bitcast_convert_type(a, jnp.int8).reshape(NPAD, HLL_M)


def kernel(x, edge_index):
    ei = edge_index.astype(jnp.int32)
    src, dst = ei[0], ei[1]
    mh0 = jnp.asarray(_MH0_NP)
    hll0 = _pack8(jnp.asarray(_HLL0_NP))
    mh1p, hll1p = _hop(mh0, hll0, src, dst)
    mh2p, hll2p = _hop(mh1p, hll1p, src, dst)
    hll1 = _unpack8(hll1p)
    hll2 = _unpack8(hll2p)
    c1, c2 = _cards(hll1, hll2)
    bias = jnp.int64(1) << 31
    mh1 = mh1p[:N_NODES].astype(jnp.int64) + bias
    mh2 = mh2p[:N_NODES].astype(jnp.int64) + bias
    cards = jnp.stack([c1[:N_NODES], c2[:N_NODES]], axis=1)
    return (x, hll1[:N_NODES], mh1, hll2[:N_NODES], mh2, cards)


# speedup vs baseline: 29.6014x; 7.7008x over previous
import functools

import numpy as np
import jax
import jax.numpy as jnp
from jax import lax
from jax.experimental import pallas as pl
from jax.experimental.pallas import tpu as pltpu
from jax.experimental.pallas import tpu_sc as plsc

N_NODES = 10000
N_TILES = 32
RPT = 320
NPAD = N_TILES * RPT
E = 320000
C = 4000
NCHUNK = E // C
NG16 = C // 16
G = 128
FBUF = 4096

NUM_PERM = 128
HLL_M = 256
MAX_RANK = 24
ALPHA = 0.7213 / (1.0 + 1.079 / HLL_M)


def _init_tables():
    rng = np.random.RandomState(1)
    perm_a = rng.randint(1, 2 ** 30, size=(1, NUM_PERM)).astype(np.int64)
    perm_b = rng.randint(0, 2 ** 30, size=(1, NUM_PERM)).astype(np.int64)
    ids = np.arange(N_NODES, dtype=np.int64)
    h = (ids * np.int64(2654435761)) & np.int64(0xFFFFFFFF)
    mh = ((perm_a * h[:, None] + perm_b) % np.int64((1 << 61) - 1)) \
        & np.int64((1 << 32) - 1)
    mh32 = (mh - np.int64(1 << 31)).astype(np.int32)
    mh32 = np.concatenate(
        [mh32, np.zeros((NPAD - N_NODES, NUM_PERM), np.int32)], axis=0)

    reg_index = (h & np.int64(HLL_M - 1)).astype(np.int64)
    bits = h >> 8
    bit_len = np.where(bits > 0,
                       np.floor(np.log2(np.maximum(bits, 1).astype(np.float64)))
                       .astype(np.int64) + 1, 0)
    ranks = (MAX_RANK - bit_len + 1).astype(np.int8)
    regs = np.zeros((NPAD, HLL_M), dtype=np.int8)
    regs[np.arange(N_NODES), reg_index] = ranks
    return mh32, regs


_MH0_NP, _HLL0_NP = _init_tables()


def _hop_body(mh_tab, hll_tab, src_hbm, dst_hbm, mh_out, hll_out,
              acc_mh, acc_hll, srcb, dstb, fsrc, fdst, rows_mh, rows_hll,
              sem, sem2):
    i32 = jnp.int32
    cid = lax.axis_index("c").astype(i32)
    sid = lax.axis_index("s").astype(i32)
    wid = sid * i32(2) + cid
    lo = wid * i32(RPT)

    pltpu.sync_copy(mh_tab.at[pl.ds(lo, RPT)], acc_mh)
    pltpu.sync_copy(hll_tab.at[pl.ds(lo, RPT)], acc_hll)

    def zfill(k, carry):
        fsrc[pl.ds(k * i32(16), 16)] = k * i32(16) + lax.iota(jnp.int32, 16)
        return carry
    lax.fori_loop(i32(0), i32(FBUF // 16), zfill, i32(0))

    def chunk_body(c, carry):
        pltpu.sync_copy(src_hbm.at[pl.ds(c * i32(C), C)], srcb)
        pltpu.sync_copy(dst_hbm.at[pl.ds(c * i32(C), C)], dstb)

        def filt(g, cnt):
            s16 = srcb[pl.ds(g * i32(16), 16)]
            d16 = dstb[pl.ds(g * i32(16), 16)]
            dl = d16 - lo
            m = (dl >= i32(0)) & (dl < i32(RPT))
            mi = m.astype(jnp.int32)
            pos = jnp.cumsum(mi) - mi + cnt
            plsc.store_scatter(fsrc, [pos], s16, mask=m)
            plsc.store_scatter(fdst, [pos], dl, mask=m)
            return cnt + jnp.sum(mi, dtype=jnp.int32)

        k_edges = lax.fori_loop(i32(0), i32(NG16), filt, i32(0))
        ngroups = (k_edges + i32(G - 1)) // i32(G)

        def grp(g, carry2):
            idx = fsrc.at[pl.ds(g * i32(G), G)]
            cp1 = pltpu.async_copy(mh_tab.at[idx], rows_mh, sem)
            cp2 = pltpu.async_copy(hll_tab.at[idx], rows_hll, sem2)
            cp1.wait()
            cp2.wait()

            def apply(i, carry3):
                d = fdst[pl.ds(i, 16)][0]
                r = i - g * i32(G)
                for sub in range(NUM_PERM // 16):
                    sl = pl.ds(sub * 16, 16)
                    acc_mh[d, sl] = jnp.minimum(acc_mh[d, sl], rows_mh[r, sl])
                for sub in range(HLL_M // 64):
                    sl = pl.ds(sub * 16, 16)
                    a8 = plsc.bitcast(acc_hll[d, sl], jnp.int8)
                    r8 = plsc.bitcast(rows_hll[r, sl], jnp.int8)
                    acc_hll[d, sl] = plsc.bitcast(jnp.maximum(a8, r8),
                                                  jnp.int32)
                return carry3

            lax.fori_loop(g * i32(G),
                          jnp.minimum(k_edges, (g + i32(1)) * i32(G)),
                          apply, i32(0))
            return carry2

        lax.fori_loop(i32(0), ngroups, grp, i32(0))
        return carry

    lax.fori_loop(i32(0), i32(NCHUNK), chunk_body, i32(0))

    pltpu.sync_copy(acc_mh, mh_out.at[pl.ds(lo, RPT)])
    pltpu.sync_copy(acc_hll, hll_out.at[pl.ds(lo, RPT)])


_hop = functools.partial(
    pl.kernel,
    mesh=plsc.VectorSubcoreMesh(core_axis_name="c", subcore_axis_name="s"),
    compiler_params=pltpu.CompilerParams(needs_layout_passes=False,
                                         use_tc_tiling_on_sc=False),
    out_type=[
        jax.ShapeDtypeStruct((NPAD, NUM_PERM), jnp.int32),
        jax.ShapeDtypeStruct((NPAD, HLL_M // 4), jnp.int32),
    ],
    scratch_types=[
        pltpu.VMEM((RPT, NUM_PERM), jnp.int32),
        pltpu.VMEM((RPT, HLL_M // 4), jnp.int32),
        pltpu.VMEM((C,), jnp.int32),
        pltpu.VMEM((C,), jnp.int32),
        pltpu.VMEM((FBUF,), jnp.int32),
        pltpu.VMEM((FBUF,), jnp.int32),
        pltpu.VMEM((G, NUM_PERM), jnp.int32),
        pltpu.VMEM((G, HLL_M // 4), jnp.int32),
        pltpu.SemaphoreType.DMA,
        pltpu.SemaphoreType.DMA,
    ],
)(_hop_body)


def _cards_body(h1_ref, h2_ref, o1_ref, o2_ref):
    for href, oref in ((h1_ref, o1_ref), (h2_ref, o2_ref)):
        r = href[...].astype(jnp.float32)
        e = ALPHA * (HLL_M * HLL_M) / jnp.sum(jnp.exp2(-r), axis=1)
        v = jnp.sum((href[...] == 0).astype(jnp.float32), axis=1)
        lc = HLL_M * jnp.log(HLL_M / jnp.maximum(v, 1.0))
        oref[...] = jnp.where((e <= 2.5 * HLL_M) & (v > 0), lc, e)


def _cards(h1, h2):
    return pl.pallas_call(
        _cards_body,
        out_shape=[
            jax.ShapeDtypeStruct((NPAD,), jnp.float32),
            jax.ShapeDtypeStruct((NPAD,), jnp.float32),
        ],
    )(h1, h2)


def _pack8(a):
    return lax.bitcast_convert_type(
        a.reshape(NPAD, HLL_M // 4, 4), jnp.int32)


def _unpack8(a):
    return lax.bitcast_convert_type(a, jnp.int8).reshape(NPAD, HLL_M)


def kernel(x, edge_index):
    ei = edge_index.astype(jnp.int32)
    src, dst = ei[0], ei[1]
    mh0 = jnp.asarray(_MH0_NP)
    hll0 = _pack8(jnp.asarray(_HLL0_NP))
    mh1p, hll1p = _hop(mh0, hll0, src, dst)
    mh2p, hll2p = _hop(mh1p, hll1p, src, dst)
    hll1 = _unpack8(hll1p)
    hll2 = _unpack8(hll2p)
    c1, c2 = _cards(hll1, hll2)
    bias = jnp.int64(1) << 31
    mh1 = mh1p[:N_NODES].astype(jnp.int64) + bias
    mh2 = mh2p[:N_NODES].astype(jnp.int64) + bias
    cards = jnp.stack([c1[:N_NODES], c2[:N_NODES]], axis=1)
    return (x, hll1[:N_NODES], mh1, hll2[:N_NODES], mh2, cards)


# speedup vs baseline: 36.9531x; 1.2484x over previous
import functools

import numpy as np
import jax
import jax.numpy as jnp
from jax import lax
from jax.experimental import pallas as pl
from jax.experimental.pallas import tpu as pltpu
from jax.experimental.pallas import tpu_sc as plsc

N_NODES = 10000
N_TILES = 32
RPT = 320
NPAD = N_TILES * RPT
E = 320000
C = 2000
NCHUNK = E // C
NG16 = C // 16
G = 80
FBUF = 2048

NUM_PERM = 128
HLL_M = 256
MAX_RANK = 24
ALPHA = 0.7213 / (1.0 + 1.079 / HLL_M)


def _init_tables():
    rng = np.random.RandomState(1)
    perm_a = rng.randint(1, 2 ** 30, size=(1, NUM_PERM)).astype(np.int64)
    perm_b = rng.randint(0, 2 ** 30, size=(1, NUM_PERM)).astype(np.int64)
    ids = np.arange(N_NODES, dtype=np.int64)
    h = (ids * np.int64(2654435761)) & np.int64(0xFFFFFFFF)
    mh = ((perm_a * h[:, None] + perm_b) % np.int64((1 << 61) - 1)) \
        & np.int64((1 << 32) - 1)
    mh32 = (mh - np.int64(1 << 31)).astype(np.int32)
    mh32 = np.concatenate(
        [mh32, np.zeros((NPAD - N_NODES, NUM_PERM), np.int32)], axis=0)

    reg_index = (h & np.int64(HLL_M - 1)).astype(np.int64)
    bits = h >> 8
    bit_len = np.where(bits > 0,
                       np.floor(np.log2(np.maximum(bits, 1).astype(np.float64)))
                       .astype(np.int64) + 1, 0)
    ranks = (MAX_RANK - bit_len + 1).astype(np.int8)
    regs = np.zeros((NPAD, HLL_M), dtype=np.int8)
    regs[np.arange(N_NODES), reg_index] = ranks
    return mh32, regs


_MH0_NP, _HLL0_NP = _init_tables()


def _hop_body(mh_tab, hll_tab, src_hbm, dst_hbm, mh_out, hll_out,
              acc_mh, acc_hll,
              srcb0, srcb1, dstb0, dstb1, fsrc0, fsrc1, fdst0, fdst1,
              rmh0, rmh1, rhll0, rhll1,
              semE0, semE1, semM0, semM1, semH0, semH1):
    i32 = jnp.int32
    cid = lax.axis_index("c").astype(i32)
    sid = lax.axis_index("s").astype(i32)
    wid = sid * i32(2) + cid
    lo = wid * i32(RPT)

    srcb = (srcb0, srcb1)
    dstb = (dstb0, dstb1)
    fsrc = (fsrc0, fsrc1)
    fdst = (fdst0, fdst1)
    rmh = (rmh0, rmh1)
    rhll = (rhll0, rhll1)
    semE = (semE0, semE1)
    semM = (semM0, semM1)
    semH = (semH0, semH1)

    pltpu.sync_copy(mh_tab.at[pl.ds(lo, RPT)], acc_mh)
    pltpu.sync_copy(hll_tab.at[pl.ds(lo, RPT)], acc_hll)

    def zfill(k, carry):
        v = k * i32(16) + lax.iota(jnp.int32, 16)
        fsrc0[pl.ds(k * i32(16), 16)] = v
        fsrc1[pl.ds(k * i32(16), 16)] = v
        return carry
    lax.fori_loop(i32(0), i32(FBUF // 16), zfill, i32(0))

    def issue_edges(c, par):
        off = pl.ds(c * i32(C), C)
        pltpu.async_copy(src_hbm.at[off], srcb[par], semE[par])
        pltpu.async_copy(dst_hbm.at[off], dstb[par], semE[par])

    def wait_edges(par):
        s = pl.ds(0, C)
        pltpu.make_async_copy(src_hbm.at[s], srcb[par], semE[par]).wait()
        pltpu.make_async_copy(dst_hbm.at[s], dstb[par], semE[par]).wait()

    def do_filter(par):
        sb, db, fs, fd = srcb[par], dstb[par], fsrc[par], fdst[par]

        def filt(g, cnt):
            s16 = sb[pl.ds(g * i32(16), 16)]
            d16 = db[pl.ds(g * i32(16), 16)]
            dl = d16 - lo
            m = (dl >= i32(0)) & (dl < i32(RPT))
            mi = m.astype(jnp.int32)
            pos = jnp.cumsum(mi) - mi + cnt
            plsc.store_scatter(fs, [pos], s16, mask=m)
            plsc.store_scatter(fd, [pos], dl, mask=m)
            return cnt + jnp.sum(mi, dtype=jnp.int32)

        return lax.fori_loop(i32(0), i32(NG16), filt, i32(0))

    def issue_gather(par, off):
        idx = fsrc[par].at[pl.ds(off, G)]
        pltpu.async_copy(mh_tab.at[idx], rmh[par], semM[par])
        pltpu.async_copy(hll_tab.at[idx], rhll[par], semH[par])

    def wait_gather(par):
        s = pl.ds(0, G)
        pltpu.make_async_copy(mh_tab.at[s], rmh[par], semM[par]).wait()
        pltpu.make_async_copy(hll_tab.at[s], rhll[par], semH[par]).wait()

    def do_apply(par, k):
        fd, rm, rh = fdst[par], rmh[par], rhll[par]

        def apply_range(base, hi_cap):
            def apply(i, carry3):
                d = fd[pl.ds(i, 16)][0]
                r = i - base
                for sub in range(NUM_PERM // 16):
                    sl = pl.ds(sub * 16, 16)
                    acc_mh[d, sl] = jnp.minimum(acc_mh[d, sl], rm[r, sl])
                for sub in range(HLL_M // 64):
                    sl = pl.ds(sub * 16, 16)
                    a8 = plsc.bitcast(acc_hll[d, sl], jnp.int8)
                    r8 = plsc.bitcast(rh[r, sl], jnp.int8)
                    acc_hll[d, sl] = plsc.bitcast(jnp.maximum(a8, r8),
                                                  jnp.int32)
                return carry3
            lax.fori_loop(base, hi_cap, apply, i32(0))

        wait_gather(par)
        apply_range(i32(0), jnp.minimum(k, i32(G)))

        ngroups = (k + i32(G - 1)) // i32(G)

        def extra(g, carry2):
            off = g * i32(G)
            issue_gather(par, off)
            wait_gather(par)
            apply_range(off, jnp.minimum(k, off + i32(G)))
            return carry2

        lax.fori_loop(i32(1), ngroups, extra, i32(0))

    def step(c, par, k_prev, issue_next_edges=True):
        wait_edges(1 - par)
        k_next = do_filter(1 - par)
        issue_gather(1 - par, i32(0))
        if issue_next_edges:
            issue_edges(c + i32(2), par)
        do_apply(par, k_prev)
        return k_next

    issue_edges(i32(0), 0)
    issue_edges(i32(1), 1)
    wait_edges(0)
    k0 = do_filter(0)
    issue_gather(0, i32(0))

    def pair(t, k_prev):
        c = t * i32(2)
        k1 = step(c, 0, k_prev)
        k2 = step(c + i32(1), 1, k1)
        return k2

    k_last = lax.fori_loop(i32(0), i32((NCHUNK - 2) // 2), pair, k0)
    k_fin = step(i32(NCHUNK - 2), 0, k_last, issue_next_edges=False)
    do_apply(1, k_fin)

    pltpu.sync_copy(acc_mh, mh_out.at[pl.ds(lo, RPT)])
    pltpu.sync_copy(acc_hll, hll_out.at[pl.ds(lo, RPT)])


_hop = functools.partial(
    pl.kernel,
    mesh=plsc.VectorSubcoreMesh(core_axis_name="c", subcore_axis_name="s"),
    compiler_params=pltpu.CompilerParams(needs_layout_passes=False,
                                         use_tc_tiling_on_sc=False),
    out_type=[
        jax.ShapeDtypeStruct((NPAD, NUM_PERM), jnp.int32),
        jax.ShapeDtypeStruct((NPAD, HLL_M // 4), jnp.int32),
    ],
    scratch_types=[
        pltpu.VMEM((RPT, NUM_PERM), jnp.int32),
        pltpu.VMEM((RPT, HLL_M // 4), jnp.int32),
        pltpu.VMEM((C,), jnp.int32),
        pltpu.VMEM((C,), jnp.int32),
        pltpu.VMEM((C,), jnp.int32),
        pltpu.VMEM((C,), jnp.int32),
        pltpu.VMEM((FBUF,), jnp.int32),
        pltpu.VMEM((FBUF,), jnp.int32),
        pltpu.VMEM((FBUF,), jnp.int32),
        pltpu.VMEM((FBUF,), jnp.int32),
        pltpu.VMEM((G, NUM_PERM), jnp.int32),
        pltpu.VMEM((G, NUM_PERM), jnp.int32),
        pltpu.VMEM((G, HLL_M // 4), jnp.int32),
        pltpu.VMEM((G, HLL_M // 4), jnp.int32),
        pltpu.SemaphoreType.DMA,
        pltpu.SemaphoreType.DMA,
        pltpu.SemaphoreType.DMA,
        pltpu.SemaphoreType.DMA,
        pltpu.SemaphoreType.DMA,
        pltpu.SemaphoreType.DMA,
    ],
)(_hop_body)


def _cards_body(h1_ref, h2_ref, o1_ref, o2_ref):
    for href, oref in ((h1_ref, o1_ref), (h2_ref, o2_ref)):
        r = href[...].astype(jnp.float32)
        e = ALPHA * (HLL_M * HLL_M) / jnp.sum(jnp.exp2(-r), axis=1)
        v = jnp.sum((href[...] == 0).astype(jnp.float32), axis=1)
        lc = HLL_M * jnp.log(HLL_M / jnp.maximum(v, 1.0))
        oref[...] = jnp.where((e <= 2.5 * HLL_M) & (v > 0), lc, e)


def _cards(h1, h2):
    return pl.pallas_call(
        _cards_body,
        out_shape=[
            jax.ShapeDtypeStruct((NPAD,), jnp.float32),
            jax.ShapeDtypeStruct((NPAD,), jnp.float32),
        ],
    )(h1, h2)


def _pack8(a):
    return lax.bitcast_convert_type(
        a.reshape(NPAD, HLL_M // 4, 4), jnp.int32)


def _unpack8(a):
    return lax.bitcast_convert_type(a, jnp.int8).reshape(NPAD, HLL_M)


def kernel(x, edge_index):
    ei = edge_index.astype(jnp.int32)
    src, dst = ei[0], ei[1]
    mh0 = jnp.asarray(_MH0_NP)
    hll0 = _pack8(jnp.asarray(_HLL0_NP))
    mh1p, hll1p = _hop(mh0, hll0, src, dst)
    mh2p, hll2p = _hop(mh1p, hll1p, src, dst)
    hll1 = _unpack8(hll1p)
    hll2 = _unpack8(hll2p)
    c1, c2 = _cards(hll1, hll2)
    bias = jnp.int64(1) << 31
    mh1 = mh1p[:N_NODES].astype(jnp.int64) + bias
    mh2 = mh2p[:N_NODES].astype(jnp.int64) + bias
    cards = jnp.stack([c1[:N_NODES], c2[:N_NODES]], axis=1)
    return (x, hll1[:N_NODES], mh1, hll2[:N_NODES], mh2, cards)
